# Initial kernel scaffold; baseline (speedup 1.0000x reference)
import jax
import jax.numpy as jnp
from jax.experimental import pallas as pl


def kernel(input_points, tables):
    raise NotImplementedError("write your pallas kernel here")


# speedup vs baseline: 20.3073x; 20.3073x over previous
import numpy as np
import jax
import jax.numpy as jnp
from jax import lax
from jax.experimental import pallas as pl
from jax.experimental.pallas import tpu as pltpu
from jax.experimental.pallas import tpu_sc as plsc

_N_LEVELS = 16
_N_FEATS = 2
_FINEST_RES = 512.0
_COARSE_RES = 16.0
_LOG2_HASH_SZ = 19
_TBL = 1 << _LOG2_HASH_SZ
_MASK = _TBL - 1
_N = 262144
_B_GROWTH = float(np.exp((np.log(_FINEST_RES) - np.log(_COARSE_RES)) / (_N_LEVELS - 1)))
_CX, _CY, _CZ = np.int32(73856093), np.int32(19349663), np.int32(83492791)

_NC, _NS, _L = 2, 16, 16
_NW = _NC * _NS
_PPW = _N // _NW
_CHUNK = 128
_NCHUNK = _PPW // _CHUNK
_G16 = _CHUNK // 16
_G8 = _CHUNK // 8
_CE = _CHUNK * 2


def _expand(v, pairh):
    lo = v.at[pairh].get(mode="promise_in_bounds")
    hi = v.at[pairh + 8].get(mode="promise_in_bounds")
    return lo, hi


def _hash_phase(pts_v, gs_v, w_v, idx_v, buf, lvl, coff):
    lanes = lax.iota(jnp.int32, 16)
    pairh = lanes >> 1
    fpair = lanes & 1
    gs = gs_v[pl.ds(lvl * 16, 16)]
    addv = fpair + lvl * (2 * _TBL)
    for g in range(_G16):
        off = coff + g * 16
        p = [pts_v[pl.ds(d * _PPW + off, 16)] for d in range(3)]
        q = [p[d] / gs for d in range(3)]
        bl = [q[d].astype(jnp.int32) for d in range(3)]
        blf = [bl[d].astype(jnp.float32) for d in range(3)]
        vmin = [blf[d] * gs for d in range(3)]
        vmax = [vmin[d] + gs for d in range(3)]
        for d in range(3):
            w = (p[d] - vmin[d]) / (vmax[d] - vmin[d])
            wlo, whi = _expand(w, pairh)
            wb = (buf * 3 + d) * _CE + g * 32
            w_v[pl.ds(wb, 16)] = wlo
            w_v[pl.ds(wb + 16, 16)] = whi
        hx = bl[0] * _CX
        hy = bl[1] * _CY
        hz = bl[2] * _CZ
        hx1 = hx + _CX
        hy1 = hy + _CY
        hz1 = hz + _CZ
        k = 0
        for hxx in (hx, hx1):
            for hyy in (hy, hy1):
                hxy = hxx ^ hyy
                for hzz in (hz, hz1):
                    h = (hxy ^ hzz) & _MASK
                    hlo, hhi = _expand(h, pairh)
                    ib = (buf * 8 + k) * _CE + g * 32
                    idx_v[pl.ds(ib, 16)] = hlo * 2 + addv
                    idx_v[pl.ds(ib + 16, 16)] = hhi * 2 + addv
                    k += 1


def _interp_phase(w_v, emb_v, out_v, buf, lvl):
    lanes = lax.iota(jnp.int32, 16)
    pairh = lanes >> 1
    fpair = lanes & 1
    outp = pairh * 32 + fpair
    obase = lvl * 2
    for g in range(_G8):
        eb = (buf * 8) * _CE + g * 16
        e = [emb_v[pl.ds(eb + k * _CE, 16)] for k in range(8)]
        wb = (buf * 3) * _CE + g * 16
        wx = w_v[pl.ds(wb, 16)]
        wy = w_v[pl.ds(wb + _CE, 16)]
        wz = w_v[pl.ds(wb + 2 * _CE, 16)]
        omx = 1.0 - wx
        omy = 1.0 - wy
        omz = 1.0 - wz
        c00 = e[0] * omx + e[4] * wx
        c01 = e[1] * omx + e[5] * wx
        c10 = e[2] * omx + e[6] * wx
        c11 = e[3] * omx + e[7] * wx
        c0 = c00 * omy + c10 * wy
        c1 = c01 * omy + c11 * wy
        c = c0 * omz + c1 * wz
        oidx = outp + (g * 8 * 32 + obase)
        plsc.store_scatter(out_v, [oidx], c)


def _fire_gathers(tab_hbm, idx_v, emb_v, buf, sem):
    descs = []
    for k in range(8):
        for h in range(2):
            s = (buf * 8 + k) * _CE + h * _CHUNK
            descs.append(pltpu.async_copy(
                tab_hbm.at[idx_v.at[pl.ds(s, _CHUNK)]],
                emb_v.at[pl.ds(s, _CHUNK)], sem))
    return descs


def _body(pts_hbm, tab_hbm, gs_hbm, out_hbm,
          pts_v, gs_v, w_v, idx_v, emb_v, out_v, sem0, sem1):
    cid = lax.axis_index("c")
    sid = lax.axis_index("s")
    wid = sid * _NC + cid
    pbase = wid * _PPW
    for d in range(3):
        pltpu.sync_copy(pts_hbm.at[pl.ds(d * _N + pbase, _PPW)],
                        pts_v.at[pl.ds(d * _PPW, _PPW)])
    pltpu.sync_copy(gs_hbm, gs_v)

    @pl.loop(0, _NCHUNK)
    def _chunk(c):
        coff = c * _CHUNK

        @pl.loop(0, _N_LEVELS // 2)
        def _lvlpair(lp):
            l0 = lp * 2
            l1 = l0 + 1
            _hash_phase(pts_v, gs_v, w_v, idx_v, 0, l0, coff)
            d0 = _fire_gathers(tab_hbm, idx_v, emb_v, 0, sem0)
            _hash_phase(pts_v, gs_v, w_v, idx_v, 1, l1, coff)
            d1 = _fire_gathers(tab_hbm, idx_v, emb_v, 1, sem1)
            for d in d0:
                d.wait()
            _interp_phase(w_v, emb_v, out_v, 0, l0)
            for d in d1:
                d.wait()
            _interp_phase(w_v, emb_v, out_v, 1, l1)

        pltpu.sync_copy(out_v, out_hbm.at[pl.ds((pbase + coff) * 32, _CHUNK * 32)])


@jax.jit
def _embed(pts_flat, tab_flat, gs_splat):
    mesh = plsc.VectorSubcoreMesh(core_axis_name="c", subcore_axis_name="s")
    f = pl.kernel(
        _body,
        out_type=jax.ShapeDtypeStruct((_N * 32,), jnp.float32),
        mesh=mesh,
        scratch_types=[
            pltpu.VMEM((3 * _PPW,), jnp.float32),
            pltpu.VMEM((_N_LEVELS * 16,), jnp.float32),
            pltpu.VMEM((2 * 3 * _CE,), jnp.float32),
            pltpu.VMEM((2 * 8 * _CE,), jnp.int32),
            pltpu.VMEM((2 * 8 * _CE,), jnp.float32),
            pltpu.VMEM((_CHUNK * 32,), jnp.float32),
            pltpu.SemaphoreType.DMA,
            pltpu.SemaphoreType.DMA,
        ],
        compiler_params=pltpu.CompilerParams(
            needs_layout_passes=False, use_tc_tiling_on_sc=False),
    )
    return f(pts_flat, tab_flat, gs_splat)


def kernel(input_points, tables):
    pts_flat = input_points.T.reshape(-1)
    tab_flat = tables.reshape(-1)
    rows = []
    for i in range(_N_LEVELS):
        res = jnp.floor(jnp.float32(_COARSE_RES) * jnp.float32(_B_GROWTH) ** i)
        gs = (jnp.float32(1.0) - jnp.float32(0.0)) / res
        rows.append(jnp.full((16,), gs, jnp.float32))
    gs_splat = jnp.concatenate(rows)
    out = _embed(pts_flat, tab_flat, gs_splat)
    return out.reshape(_N, 32)


# speedup vs baseline: 20.4612x; 1.0076x over previous
import numpy as np
import jax
import jax.numpy as jnp
from jax import lax
from jax.experimental import pallas as pl
from jax.experimental.pallas import tpu as pltpu
from jax.experimental.pallas import tpu_sc as plsc

_N_LEVELS = 16
_N_FEATS = 2
_FINEST_RES = 512.0
_COARSE_RES = 16.0
_LOG2_HASH_SZ = 19
_TBL = 1 << _LOG2_HASH_SZ
_MASK = _TBL - 1
_N = 262144
_B_GROWTH = float(np.exp((np.log(_FINEST_RES) - np.log(_COARSE_RES)) / (_N_LEVELS - 1)))
_CX, _CY, _CZ = np.int32(73856093), np.int32(19349663), np.int32(83492791)

_NC, _NS, _L = 2, 16, 16
_NW = _NC * _NS
_PPW = _N // _NW
_CHUNK = 128
_NCHUNK = _PPW // _CHUNK
_G16 = _CHUNK // 16
_G8 = _CHUNK // 8
_CE = _CHUNK * 2


def _expand(v, pairh):
    lo = v.at[pairh].get(mode="promise_in_bounds")
    hi = v.at[pairh + 8].get(mode="promise_in_bounds")
    return lo, hi


def _hash_phase(pts_v, gs_v, w_v, idx_v, buf, lvl, coff):
    lanes = lax.iota(jnp.int32, 16)
    pairh = lanes >> 1
    fpair = lanes & 1
    gs = gs_v[pl.ds(lvl * 16, 16)]
    addv = fpair + lvl * (2 * _TBL)
    for g in range(_G16):
        off = coff + g * 16
        p = [pts_v[pl.ds(d * _PPW + off, 16)] for d in range(3)]
        q = [p[d] / gs for d in range(3)]
        bl = [q[d].astype(jnp.int32) for d in range(3)]
        blf = [bl[d].astype(jnp.float32) for d in range(3)]
        vmin = [blf[d] * gs for d in range(3)]
        vmax = [vmin[d] + gs for d in range(3)]
        for d in range(3):
            w = (p[d] - vmin[d]) / (vmax[d] - vmin[d])
            wlo, whi = _expand(w, pairh)
            wb = (buf * 3 + d) * _CE + g * 32
            w_v[pl.ds(wb, 16)] = wlo
            w_v[pl.ds(wb + 16, 16)] = whi
        hx = bl[0] * _CX
        hy = bl[1] * _CY
        hz = bl[2] * _CZ
        hx1 = hx + _CX
        hy1 = hy + _CY
        hz1 = hz + _CZ
        k = 0
        for hxx in (hx, hx1):
            for hyy in (hy, hy1):
                hxy = hxx ^ hyy
                for hzz in (hz, hz1):
                    h = (hxy ^ hzz) & _MASK
                    hlo, hhi = _expand(h, pairh)
                    ib = (buf * 8 + k) * _CE + g * 32
                    idx_v[pl.ds(ib, 16)] = hlo * 2 + addv
                    idx_v[pl.ds(ib + 16, 16)] = hhi * 2 + addv
                    k += 1


def _interp_phase(w_v, emb_v, out_v, buf, lvl):
    lanes = lax.iota(jnp.int32, 16)
    pairh = lanes >> 1
    fpair = lanes & 1
    outp = pairh * 32 + fpair
    obase = lvl * 2
    for g in range(_G8):
        eb = (buf * 8) * _CE + g * 16
        e = [emb_v[pl.ds(eb + k * _CE, 16)] for k in range(8)]
        wb = (buf * 3) * _CE + g * 16
        wx = w_v[pl.ds(wb, 16)]
        wy = w_v[pl.ds(wb + _CE, 16)]
        wz = w_v[pl.ds(wb + 2 * _CE, 16)]
        omx = 1.0 - wx
        omy = 1.0 - wy
        omz = 1.0 - wz
        c00 = e[0] * omx + e[4] * wx
        c01 = e[1] * omx + e[5] * wx
        c10 = e[2] * omx + e[6] * wx
        c11 = e[3] * omx + e[7] * wx
        c0 = c00 * omy + c10 * wy
        c1 = c01 * omy + c11 * wy
        c = c0 * omz + c1 * wz
        oidx = outp + (g * 8 * 32 + obase)
        plsc.store_scatter(out_v, [oidx], c)


def _fire_gathers(tab_hbm, idx_v, emb_v, buf, sem):
    s = buf * 8 * _CE
    return [pltpu.async_copy(tab_hbm.at[idx_v.at[pl.ds(s, 8 * _CE)]],
                             emb_v.at[pl.ds(s, 8 * _CE)], sem)]


def _body(pts_hbm, tab_hbm, gs_hbm, out_hbm,
          pts_v, gs_v, w_v, idx_v, emb_v, out_v, sem0, sem1):
    cid = lax.axis_index("c")
    sid = lax.axis_index("s")
    wid = sid * _NC + cid
    pbase = wid * _PPW
    for d in range(3):
        pltpu.sync_copy(pts_hbm.at[pl.ds(d * _N + pbase, _PPW)],
                        pts_v.at[pl.ds(d * _PPW, _PPW)])
    pltpu.sync_copy(gs_hbm, gs_v)

    @pl.loop(0, _NCHUNK)
    def _chunk(c):
        coff = c * _CHUNK

        @pl.loop(0, _N_LEVELS // 2)
        def _lvlpair(lp):
            l0 = lp * 2
            l1 = l0 + 1
            _hash_phase(pts_v, gs_v, w_v, idx_v, 0, l0, coff)
            d0 = _fire_gathers(tab_hbm, idx_v, emb_v, 0, sem0)
            _hash_phase(pts_v, gs_v, w_v, idx_v, 1, l1, coff)
            d1 = _fire_gathers(tab_hbm, idx_v, emb_v, 1, sem1)
            for d in d0:
                d.wait()
            _interp_phase(w_v, emb_v, out_v, 0, l0)
            for d in d1:
                d.wait()
            _interp_phase(w_v, emb_v, out_v, 1, l1)

        pltpu.sync_copy(out_v, out_hbm.at[pl.ds((pbase + coff) * 32, _CHUNK * 32)])


@jax.jit
def _embed(pts_flat, tab_flat, gs_splat):
    mesh = plsc.VectorSubcoreMesh(core_axis_name="c", subcore_axis_name="s")
    f = pl.kernel(
        _body,
        out_type=jax.ShapeDtypeStruct((_N * 32,), jnp.float32),
        mesh=mesh,
        scratch_types=[
            pltpu.VMEM((3 * _PPW,), jnp.float32),
            pltpu.VMEM((_N_LEVELS * 16,), jnp.float32),
            pltpu.VMEM((2 * 3 * _CE,), jnp.float32),
            pltpu.VMEM((2 * 8 * _CE,), jnp.int32),
            pltpu.VMEM((2 * 8 * _CE,), jnp.float32),
            pltpu.VMEM((_CHUNK * 32,), jnp.float32),
            pltpu.SemaphoreType.DMA,
            pltpu.SemaphoreType.DMA,
        ],
        compiler_params=pltpu.CompilerParams(
            needs_layout_passes=False, use_tc_tiling_on_sc=False),
    )
    return f(pts_flat, tab_flat, gs_splat)


def kernel(input_points, tables):
    pts_flat = input_points.T.reshape(-1)
    tab_flat = tables.reshape(-1)
    rows = []
    for i in range(_N_LEVELS):
        res = jnp.floor(jnp.float32(_COARSE_RES) * jnp.float32(_B_GROWTH) ** i)
        gs = (jnp.float32(1.0) - jnp.float32(0.0)) / res
        rows.append(jnp.full((16,), gs, jnp.float32))
    gs_splat = jnp.concatenate(rows)
    out = _embed(pts_flat, tab_flat, gs_splat)
    return out.reshape(_N, 32)


# speedup vs baseline: 20.4677x; 1.0003x over previous
import numpy as np
import jax
import jax.numpy as jnp
from jax import lax
from jax.experimental import pallas as pl
from jax.experimental.pallas import tpu as pltpu
from jax.experimental.pallas import tpu_sc as plsc

_N_LEVELS = 16
_N_FEATS = 2
_FINEST_RES = 512.0
_COARSE_RES = 16.0
_LOG2_HASH_SZ = 19
_TBL = 1 << _LOG2_HASH_SZ
_MASK = _TBL - 1
_N = 262144
_B_GROWTH = float(np.exp((np.log(_FINEST_RES) - np.log(_COARSE_RES)) / (_N_LEVELS - 1)))
_CX, _CY, _CZ = np.int32(73856093), np.int32(19349663), np.int32(83492791)

_NC, _NS, _L = 2, 16, 16
_NW = _NC * _NS
_PPW = _N // _NW
_CHUNK = 128
_NCHUNK = _PPW // _CHUNK
_G16 = _CHUNK // 16
_G8 = _CHUNK // 8
_CE = _CHUNK * 2


def _expand(v, pairh):
    lo = v.at[pairh].get(mode="promise_in_bounds")
    hi = v.at[pairh + 8].get(mode="promise_in_bounds")
    return lo, hi


def _hash_phase(pts_v, gs_v, w_v, idx_v, buf, lvl, coff):
    lanes = lax.iota(jnp.int32, 16)
    pairh = lanes >> 1
    fpair = lanes & 1
    gs = gs_v[pl.ds(lvl * 16, 16)]
    addv = fpair + lvl * (2 * _TBL)
    for g in range(_G16):
        off = coff + g * 16
        p = [pts_v[pl.ds(d * _PPW + off, 16)] for d in range(3)]
        q = [p[d] / gs for d in range(3)]
        bl = [q[d].astype(jnp.int32) for d in range(3)]
        blf = [bl[d].astype(jnp.float32) for d in range(3)]
        vmin = [blf[d] * gs for d in range(3)]
        vmax = [vmin[d] + gs for d in range(3)]
        for d in range(3):
            w = (p[d] - vmin[d]) / (vmax[d] - vmin[d])
            wlo, whi = _expand(w, pairh)
            wb = (buf * 3 + d) * _CE + g * 32
            w_v[pl.ds(wb, 16)] = wlo
            w_v[pl.ds(wb + 16, 16)] = whi
        hx = bl[0] * _CX
        hy = bl[1] * _CY
        hz = bl[2] * _CZ
        hx1 = hx + _CX
        hy1 = hy + _CY
        hz1 = hz + _CZ
        k = 0
        for hxx in (hx, hx1):
            for hyy in (hy, hy1):
                hxy = hxx ^ hyy
                for hzz in (hz, hz1):
                    h = (hxy ^ hzz) & _MASK
                    hlo, hhi = _expand(h, pairh)
                    ib = (buf * 8 + k) * _CE + g * 32
                    idx_v[pl.ds(ib, 16)] = hlo * 2 + addv
                    idx_v[pl.ds(ib + 16, 16)] = hhi * 2 + addv
                    k += 1


def _interp_phase(w_v, emb_v, out_v, buf, lvl):
    lanes = lax.iota(jnp.int32, 16)
    pairh = lanes >> 1
    fpair = lanes & 1
    outp = pairh * 32 + fpair
    obase = lvl * 2
    for g in range(_G8):
        eb = (buf * 8) * _CE + g * 16
        e = [emb_v[pl.ds(eb + k * _CE, 16)] for k in range(8)]
        wb = (buf * 3) * _CE + g * 16
        wx = w_v[pl.ds(wb, 16)]
        wy = w_v[pl.ds(wb + _CE, 16)]
        wz = w_v[pl.ds(wb + 2 * _CE, 16)]
        omx = 1.0 - wx
        omy = 1.0 - wy
        omz = 1.0 - wz
        c00 = e[0] * omx + e[4] * wx
        c01 = e[1] * omx + e[5] * wx
        c10 = e[2] * omx + e[6] * wx
        c11 = e[3] * omx + e[7] * wx
        c0 = c00 * omy + c10 * wy
        c1 = c01 * omy + c11 * wy
        c = c0 * omz + c1 * wz
        oidx = outp + (g * 8 * 32 + obase)
        plsc.store_scatter(out_v, [oidx], c)


def _fire_gathers(tab_hbm, idx_v, emb_v, buf, sem):
    s = buf * 8 * _CE
    return [pltpu.async_copy(tab_hbm.at[idx_v.at[pl.ds(s, 8 * _CE)]],
                             emb_v.at[pl.ds(s, 8 * _CE)], sem)]


def _body(pts_hbm, tab_hbm, gs_hbm, out_hbm,
          pts_v, gs_v, w_v, idx_v, emb_v, out_v, sem0, sem1):
    cid = lax.axis_index("c")
    sid = lax.axis_index("s")
    wid = sid * _NC + cid
    pbase = wid * _PPW
    for d in range(3):
        pltpu.sync_copy(pts_hbm.at[pl.ds(d * _N + pbase, _PPW)],
                        pts_v.at[pl.ds(d * _PPW, _PPW)])
    pltpu.sync_copy(gs_hbm, gs_v)

    @pl.loop(0, _NCHUNK)
    def _chunk(c):
        coff = c * _CHUNK

        @pl.loop(0, _N_LEVELS // 2)
        def _lvlpair(lp):
            l0 = lp * 2
            l1 = l0 + 1
            _hash_phase(pts_v, gs_v, w_v, idx_v, 0, l0, coff)
            d0 = _fire_gathers(tab_hbm, idx_v, emb_v, 0, sem0)
            _hash_phase(pts_v, gs_v, w_v, idx_v, 1, l1, coff)
            d1 = _fire_gathers(tab_hbm, idx_v, emb_v, 1, sem1)
            for d in d0:
                d.wait()
            _interp_phase(w_v, emb_v, out_v, 0, l0)
            for d in d1:
                d.wait()
            _interp_phase(w_v, emb_v, out_v, 1, l1)

        pltpu.sync_copy(out_v, out_hbm.at[pl.ds((pbase + coff) * 32, _CHUNK * 32)])


@jax.jit
def _embed(pts_flat, tab_flat, gs_splat):
    mesh = plsc.VectorSubcoreMesh(core_axis_name="c", subcore_axis_name="s")
    f = pl.kernel(
        _body,
        out_type=jax.ShapeDtypeStruct((_N * 32,), jnp.float32),
        mesh=mesh,
        scratch_types=[
            pltpu.VMEM((3 * _PPW,), jnp.float32),
            pltpu.VMEM((_N_LEVELS * 16,), jnp.float32),
            pltpu.VMEM((2 * 3 * _CE,), jnp.float32),
            pltpu.VMEM((2 * 8 * _CE,), jnp.int32),
            pltpu.VMEM((2 * 8 * _CE,), jnp.float32),
            pltpu.VMEM((_CHUNK * 32,), jnp.float32),
            pltpu.SemaphoreType.DMA,
            pltpu.SemaphoreType.DMA,
        ],
        compiler_params=pltpu.CompilerParams(
            needs_layout_passes=False, use_tc_tiling_on_sc=False),
    )
    return f(pts_flat, tab_flat, gs_splat)


def kernel(input_points, tables):
    pts_flat = input_points.T.reshape(-1)
    tab2d = tables.reshape(_N_LEVELS * _TBL * _N_FEATS // 128, 128)
    tab2d = jax.lax.optimization_barrier(tab2d)
    tab_flat = tab2d.reshape(-1)
    rows = []
    for i in range(_N_LEVELS):
        res = jnp.floor(jnp.float32(_COARSE_RES) * jnp.float32(_B_GROWTH) ** i)
        gs = (jnp.float32(1.0) - jnp.float32(0.0)) / res
        rows.append(jnp.full((16,), gs, jnp.float32))
    gs_splat = jnp.concatenate(rows)
    out = _embed(pts_flat, tab_flat, gs_splat)
    return out.reshape(_N, 32)


# speedup vs baseline: 96.2255x; 4.7013x over previous
import numpy as np
import jax
import jax.numpy as jnp
from jax import lax
from jax.experimental import pallas as pl
from jax.experimental.pallas import tpu as pltpu
from jax.experimental.pallas import tpu_sc as plsc

_N_LEVELS = 16
_N_FEATS = 2
_FINEST_RES = 512.0
_COARSE_RES = 16.0
_LOG2_HASH_SZ = 19
_TBL = 1 << _LOG2_HASH_SZ
_MASK = _TBL - 1
_N = 262144
_B_GROWTH = float(np.exp((np.log(_FINEST_RES) - np.log(_COARSE_RES)) / (_N_LEVELS - 1)))
_CX, _CY, _CZ = np.int32(73856093), np.int32(19349663), np.int32(83492791)

_NC, _NS, _L = 2, 16, 16
_NW = _NC * _NS
_PPW = _N // _NW
_CHUNK = 128
_NCHUNK = _PPW // _CHUNK
_G16 = _CHUNK // 16
_G8 = _CHUNK // 8
_CE = _CHUNK * 2


def _expand(v, pairh):
    lo = v.at[pairh].get(mode="promise_in_bounds")
    hi = v.at[pairh + 8].get(mode="promise_in_bounds")
    return lo, hi


def _hash_phase(pts_v, gs_v, w_v, idx_v, buf, lvl, coff):
    lanes = lax.iota(jnp.int32, 16)
    pairh = lanes >> 1
    fpair = lanes & 1
    gs = gs_v[pl.ds(lvl * 16, 16)]
    addv = fpair * 128 + lvl * (_N_FEATS * _TBL)
    for g in range(_G16):
        off = coff + g * 16
        p = [pts_v[pl.ds(d * _PPW + off, 16)] for d in range(3)]
        q = [p[d] / gs for d in range(3)]
        bl = [q[d].astype(jnp.int32) for d in range(3)]
        blf = [bl[d].astype(jnp.float32) for d in range(3)]
        vmin = [blf[d] * gs for d in range(3)]
        vmax = [vmin[d] + gs for d in range(3)]
        for d in range(3):
            w = (p[d] - vmin[d]) / (vmax[d] - vmin[d])
            wlo, whi = _expand(w, pairh)
            wb = (buf * 3 + d) * _CE + g * 32
            w_v[pl.ds(wb, 16)] = wlo
            w_v[pl.ds(wb + 16, 16)] = whi
        hx = bl[0] * _CX
        hy = bl[1] * _CY
        hz = bl[2] * _CZ
        hx1 = hx + _CX
        hy1 = hy + _CY
        hz1 = hz + _CZ
        k = 0
        for hxx in (hx, hx1):
            for hyy in (hy, hy1):
                hxy = hxx ^ hyy
                for hzz in (hz, hz1):
                    h = (hxy ^ hzz) & _MASK
                    hlo, hhi = _expand(h, pairh)
                    ib = (buf * 8 + k) * _CE + g * 32
                    idx_v[pl.ds(ib, 16)] = hlo + (hlo & -128) + addv
                    idx_v[pl.ds(ib + 16, 16)] = hhi + (hhi & -128) + addv
                    k += 1


def _interp_phase(w_v, emb_v, out_v, buf, lvl):
    lanes = lax.iota(jnp.int32, 16)
    pairh = lanes >> 1
    fpair = lanes & 1
    outp = pairh * 32 + fpair
    obase = lvl * 2
    for g in range(_G8):
        eb = (buf * 8) * _CE + g * 16
        e = [emb_v[pl.ds(eb + k * _CE, 16)] for k in range(8)]
        wb = (buf * 3) * _CE + g * 16
        wx = w_v[pl.ds(wb, 16)]
        wy = w_v[pl.ds(wb + _CE, 16)]
        wz = w_v[pl.ds(wb + 2 * _CE, 16)]
        omx = 1.0 - wx
        omy = 1.0 - wy
        omz = 1.0 - wz
        c00 = e[0] * omx + e[4] * wx
        c01 = e[1] * omx + e[5] * wx
        c10 = e[2] * omx + e[6] * wx
        c11 = e[3] * omx + e[7] * wx
        c0 = c00 * omy + c10 * wy
        c1 = c01 * omy + c11 * wy
        c = c0 * omz + c1 * wz
        oidx = outp + (g * 8 * 32 + obase)
        plsc.store_scatter(out_v, [oidx], c)


def _fire_gathers(tab_hbm, idx_v, emb_v, buf, sem):
    s = buf * 8 * _CE
    return [pltpu.async_copy(tab_hbm.at[idx_v.at[pl.ds(s, 8 * _CE)]],
                             emb_v.at[pl.ds(s, 8 * _CE)], sem)]


def _body(pts_hbm, tab_hbm, gs_hbm, out_hbm,
          pts_v, gs_v, w_v, idx_v, emb_v, out_v, sem0, sem1):
    cid = lax.axis_index("c")
    sid = lax.axis_index("s")
    wid = sid * _NC + cid
    pbase = wid * _PPW
    for d in range(3):
        pltpu.sync_copy(pts_hbm.at[pl.ds(d * _N + pbase, _PPW)],
                        pts_v.at[pl.ds(d * _PPW, _PPW)])
    pltpu.sync_copy(gs_hbm, gs_v)

    @pl.loop(0, _NCHUNK)
    def _chunk(c):
        coff = c * _CHUNK

        @pl.loop(0, _N_LEVELS // 2)
        def _lvlpair(lp):
            l0 = lp * 2
            l1 = l0 + 1
            _hash_phase(pts_v, gs_v, w_v, idx_v, 0, l0, coff)
            d0 = _fire_gathers(tab_hbm, idx_v, emb_v, 0, sem0)
            _hash_phase(pts_v, gs_v, w_v, idx_v, 1, l1, coff)
            d1 = _fire_gathers(tab_hbm, idx_v, emb_v, 1, sem1)
            for d in d0:
                d.wait()
            _interp_phase(w_v, emb_v, out_v, 0, l0)
            for d in d1:
                d.wait()
            _interp_phase(w_v, emb_v, out_v, 1, l1)

        pltpu.sync_copy(out_v, out_hbm.at[pl.ds((pbase + coff) * 32, _CHUNK * 32)])


@jax.jit
def _embed(pts_flat, tab_flat, gs_splat):
    mesh = plsc.VectorSubcoreMesh(core_axis_name="c", subcore_axis_name="s")
    f = pl.kernel(
        _body,
        out_type=jax.ShapeDtypeStruct((_N * 32,), jnp.float32),
        mesh=mesh,
        scratch_types=[
            pltpu.VMEM((3 * _PPW,), jnp.float32),
            pltpu.VMEM((_N_LEVELS * 16,), jnp.float32),
            pltpu.VMEM((2 * 3 * _CE,), jnp.float32),
            pltpu.VMEM((2 * 8 * _CE,), jnp.int32),
            pltpu.VMEM((2 * 8 * _CE,), jnp.float32),
            pltpu.VMEM((_CHUNK * 32,), jnp.float32),
            pltpu.SemaphoreType.DMA,
            pltpu.SemaphoreType.DMA,
        ],
        compiler_params=pltpu.CompilerParams(
            needs_layout_passes=False, use_tc_tiling_on_sc=False),
    )
    return f(pts_flat, tab_flat, gs_splat)


def kernel(input_points, tables):
    pts_flat = input_points.T.reshape(-1)
    tab_flat = (tables.reshape(_N_LEVELS, _TBL // 128, 128, _N_FEATS)
                .transpose(0, 1, 3, 2).reshape(-1))
    rows = []
    for i in range(_N_LEVELS):
        res = jnp.floor(jnp.float32(_COARSE_RES) * jnp.float32(_B_GROWTH) ** i)
        gs = (jnp.float32(1.0) - jnp.float32(0.0)) / res
        rows.append(jnp.full((16,), gs, jnp.float32))
    gs_splat = jnp.concatenate(rows)
    out = _embed(pts_flat, tab_flat, gs_splat)
    return out.reshape(_N, 32)


# speedup vs baseline: 96.4010x; 1.0018x over previous
import numpy as np
import jax
import jax.numpy as jnp
from jax import lax
from jax.experimental import pallas as pl
from jax.experimental.pallas import tpu as pltpu
from jax.experimental.pallas import tpu_sc as plsc

_N_LEVELS = 16
_N_FEATS = 2
_FINEST_RES = 512.0
_COARSE_RES = 16.0
_LOG2_HASH_SZ = 19
_TBL = 1 << _LOG2_HASH_SZ
_MASK = _TBL - 1
_N = 262144
_B_GROWTH = float(np.exp((np.log(_FINEST_RES) - np.log(_COARSE_RES)) / (_N_LEVELS - 1)))
_CX, _CY, _CZ = np.int32(73856093), np.int32(19349663), np.int32(83492791)

_NC, _NS, _L = 2, 16, 16
_NW = _NC * _NS
_PPW = _N // _NW
_CHUNK = 128
_NCHUNK = _PPW // _CHUNK
_G16 = _CHUNK // 16
_G8 = _CHUNK // 8
_CE = _CHUNK * 2


def _expand(v, pairh):
    lo = v.at[pairh].get(mode="promise_in_bounds")
    hi = v.at[pairh + 8].get(mode="promise_in_bounds")
    return lo, hi


def _hash_phase(pts_v, gs_v, w_v, idx_v, buf, lvl, coff):
    lanes = lax.iota(jnp.int32, 16)
    pairh = lanes >> 1
    fpair = lanes & 1
    gs = gs_v[pl.ds(lvl * 16, 16)]
    rs = gs_v[pl.ds((16 + lvl) * 16, 16)]
    addv = fpair * 128 + lvl * (_N_FEATS * _TBL)
    for g in range(_G16):
        off = coff + g * 16
        p = [pts_v[pl.ds(d * _PPW + off, 16)] for d in range(3)]
        q = [p[d] * rs for d in range(3)]
        bl = [q[d].astype(jnp.int32) for d in range(3)]
        blf = [bl[d].astype(jnp.float32) for d in range(3)]
        vmin = [blf[d] * gs for d in range(3)]
        for d in range(3):
            w = (p[d] - vmin[d]) * rs
            wlo, whi = _expand(w, pairh)
            wb = (buf * 3 + d) * _CE + g * 32
            w_v[pl.ds(wb, 16)] = wlo
            w_v[pl.ds(wb + 16, 16)] = whi
        hx = bl[0] * _CX
        hy = bl[1] * _CY
        hz = bl[2] * _CZ
        hx1 = hx + _CX
        hy1 = hy + _CY
        hz1 = hz + _CZ
        k = 0
        for hxx in (hx, hx1):
            for hyy in (hy, hy1):
                hxy = hxx ^ hyy
                for hzz in (hz, hz1):
                    h = (hxy ^ hzz) & _MASK
                    hlo, hhi = _expand(h, pairh)
                    ib = (buf * 8 + k) * _CE + g * 32
                    idx_v[pl.ds(ib, 16)] = hlo + (hlo & -128) + addv
                    idx_v[pl.ds(ib + 16, 16)] = hhi + (hhi & -128) + addv
                    k += 1


def _interp_phase(w_v, emb_v, out_v, buf, lvl):
    lanes = lax.iota(jnp.int32, 16)
    pairh = lanes >> 1
    fpair = lanes & 1
    outp = pairh * 32 + fpair
    obase = lvl * 2
    for g in range(_G8):
        eb = (buf * 8) * _CE + g * 16
        e = [emb_v[pl.ds(eb + k * _CE, 16)] for k in range(8)]
        wb = (buf * 3) * _CE + g * 16
        wx = w_v[pl.ds(wb, 16)]
        wy = w_v[pl.ds(wb + _CE, 16)]
        wz = w_v[pl.ds(wb + 2 * _CE, 16)]
        omx = 1.0 - wx
        omy = 1.0 - wy
        omz = 1.0 - wz
        c00 = e[0] * omx + e[4] * wx
        c01 = e[1] * omx + e[5] * wx
        c10 = e[2] * omx + e[6] * wx
        c11 = e[3] * omx + e[7] * wx
        c0 = c00 * omy + c10 * wy
        c1 = c01 * omy + c11 * wy
        c = c0 * omz + c1 * wz
        oidx = outp + (g * 8 * 32 + obase)
        plsc.store_scatter(out_v, [oidx], c)


def _fire_gathers(tab_hbm, idx_v, emb_v, buf, sem):
    s = buf * 8 * _CE
    return [pltpu.async_copy(tab_hbm.at[idx_v.at[pl.ds(s, 8 * _CE)]],
                             emb_v.at[pl.ds(s, 8 * _CE)], sem)]


def _body(pts_hbm, tab_hbm, gs_hbm, out_hbm,
          pts_v, gs_v, w_v, idx_v, emb_v, out_v, sem0, sem1):
    cid = lax.axis_index("c")
    sid = lax.axis_index("s")
    wid = sid * _NC + cid
    pbase = wid * _PPW
    for d in range(3):
        pltpu.sync_copy(pts_hbm.at[pl.ds(d * _N + pbase, _PPW)],
                        pts_v.at[pl.ds(d * _PPW, _PPW)])
    pltpu.sync_copy(gs_hbm, gs_v)

    @pl.loop(0, _NCHUNK)
    def _chunk(c):
        coff = c * _CHUNK

        @pl.loop(0, _N_LEVELS // 2)
        def _lvlpair(lp):
            l0 = lp * 2
            l1 = l0 + 1
            _hash_phase(pts_v, gs_v, w_v, idx_v, 0, l0, coff)
            d0 = _fire_gathers(tab_hbm, idx_v, emb_v, 0, sem0)
            _hash_phase(pts_v, gs_v, w_v, idx_v, 1, l1, coff)
            d1 = _fire_gathers(tab_hbm, idx_v, emb_v, 1, sem1)
            for d in d0:
                d.wait()
            _interp_phase(w_v, emb_v, out_v, 0, l0)
            for d in d1:
                d.wait()
            _interp_phase(w_v, emb_v, out_v, 1, l1)

        pltpu.sync_copy(out_v, out_hbm.at[pl.ds((pbase + coff) * 32, _CHUNK * 32)])


@jax.jit
def _embed(pts_flat, tab_flat, gs_splat):
    mesh = plsc.VectorSubcoreMesh(core_axis_name="c", subcore_axis_name="s")
    f = pl.kernel(
        _body,
        out_type=jax.ShapeDtypeStruct((_N * 32,), jnp.float32),
        mesh=mesh,
        scratch_types=[
            pltpu.VMEM((3 * _PPW,), jnp.float32),
            pltpu.VMEM((2 * _N_LEVELS * 16,), jnp.float32),
            pltpu.VMEM((2 * 3 * _CE,), jnp.float32),
            pltpu.VMEM((2 * 8 * _CE,), jnp.int32),
            pltpu.VMEM((2 * 8 * _CE,), jnp.float32),
            pltpu.VMEM((_CHUNK * 32,), jnp.float32),
            pltpu.SemaphoreType.DMA,
            pltpu.SemaphoreType.DMA,
        ],
        compiler_params=pltpu.CompilerParams(
            needs_layout_passes=False, use_tc_tiling_on_sc=False),
    )
    return f(pts_flat, tab_flat, gs_splat)


def kernel(input_points, tables):
    pts_flat = input_points.T.reshape(-1)
    tab_flat = (tables.reshape(_N_LEVELS, _TBL // 128, 128, _N_FEATS)
                .transpose(0, 1, 3, 2).reshape(-1))
    rows = []
    rrows = []
    for i in range(_N_LEVELS):
        res = jnp.floor(jnp.float32(_COARSE_RES) * jnp.float32(_B_GROWTH) ** i)
        gs = (jnp.float32(1.0) - jnp.float32(0.0)) / res
        rows.append(jnp.full((16,), gs, jnp.float32))
        rrows.append(jnp.full((16,), res, jnp.float32))
    gs_splat = jnp.concatenate(rows + rrows)
    out = _embed(pts_flat, tab_flat, gs_splat)
    return out.reshape(_N, 32)


# speedup vs baseline: 106.4937x; 1.1047x over previous
import numpy as np
import jax
import jax.numpy as jnp
from jax import lax
from jax.experimental import pallas as pl
from jax.experimental.pallas import tpu as pltpu
from jax.experimental.pallas import tpu_sc as plsc

_N_LEVELS = 16
_N_FEATS = 2
_FINEST_RES = 512.0
_COARSE_RES = 16.0
_LOG2_HASH_SZ = 19
_TBL = 1 << _LOG2_HASH_SZ
_MASK = _TBL - 1
_N = 262144
_B_GROWTH = float(np.exp((np.log(_FINEST_RES) - np.log(_COARSE_RES)) / (_N_LEVELS - 1)))
_CX, _CY, _CZ = np.int32(73856093), np.int32(19349663), np.int32(83492791)

_NC, _NS, _L = 2, 16, 16
_NW = _NC * _NS
_PPW = _N // _NW
_CHUNK = 128
_NCHUNK = _PPW // _CHUNK
_G16 = _CHUNK // 16
_G8 = _CHUNK // 8
_CE = _CHUNK * 2


def _expand(v, pairh):
    lo = v.at[pairh].get(mode="promise_in_bounds")
    hi = v.at[pairh + 8].get(mode="promise_in_bounds")
    return lo, hi


def _hash_phase(pts_v, gs_v, w_v, idx_v, buf, lvl, coff):
    lanes = lax.iota(jnp.int32, 16)
    pairh = lanes >> 1
    fpair = lanes & 1
    gs = gs_v[pl.ds(lvl * 16, 16)]
    rs = gs_v[pl.ds((16 + lvl) * 16, 16)]
    addv = fpair * 128 + lvl * (_N_FEATS * _TBL)
    for g in range(_G16):
        off = coff + g * 16
        p = [pts_v[pl.ds(d * _PPW + off, 16)] for d in range(3)]
        q = [p[d] * rs for d in range(3)]
        bl = [q[d].astype(jnp.int32) for d in range(3)]
        blf = [bl[d].astype(jnp.float32) for d in range(3)]
        vmin = [blf[d] * gs for d in range(3)]
        for d in range(3):
            w = (p[d] - vmin[d]) * rs
            wlo, whi = _expand(w, pairh)
            wb = (buf * 3 + d) * _CE + g * 32
            w_v[pl.ds(wb, 16)] = wlo
            w_v[pl.ds(wb + 16, 16)] = whi
        hx = bl[0] * _CX
        hy = bl[1] * _CY
        hz = bl[2] * _CZ
        hx1 = hx + _CX
        hy1 = hy + _CY
        hz1 = hz + _CZ
        k = 0
        for hxx in (hx, hx1):
            for hyy in (hy, hy1):
                hxy = hxx ^ hyy
                for hzz in (hz, hz1):
                    h = (hxy ^ hzz) & _MASK
                    hlo, hhi = _expand(h, pairh)
                    ib = (buf * 8 + k) * _CE + g * 32
                    idx_v[pl.ds(ib, 16)] = hlo + (hlo & -128) + addv
                    idx_v[pl.ds(ib + 16, 16)] = hhi + (hhi & -128) + addv
                    k += 1


def _interp_phase(w_v, emb_v, out_v, buf, lvl):
    lanes = lax.iota(jnp.int32, 16)
    pairh = lanes >> 1
    fpair = lanes & 1
    outp = pairh * 32 + fpair
    obase = lvl * 2
    for g in range(_G8):
        eb = (buf * 8) * _CE + g * 16
        e = [emb_v[pl.ds(eb + k * _CE, 16)] for k in range(8)]
        wb = (buf * 3) * _CE + g * 16
        wx = w_v[pl.ds(wb, 16)]
        wy = w_v[pl.ds(wb + _CE, 16)]
        wz = w_v[pl.ds(wb + 2 * _CE, 16)]
        omx = 1.0 - wx
        omy = 1.0 - wy
        omz = 1.0 - wz
        c00 = e[0] * omx + e[4] * wx
        c01 = e[1] * omx + e[5] * wx
        c10 = e[2] * omx + e[6] * wx
        c11 = e[3] * omx + e[7] * wx
        c0 = c00 * omy + c10 * wy
        c1 = c01 * omy + c11 * wy
        c = c0 * omz + c1 * wz
        oidx = outp + (g * 8 * 32 + obase)
        plsc.store_scatter(out_v, [oidx], c)


def _fire_gathers(tab_hbm, idx_v, emb_v, buf, sem):
    s = buf * 8 * _CE
    return [pltpu.async_copy(tab_hbm.at[idx_v.at[pl.ds(s, 8 * _CE)]],
                             emb_v.at[pl.ds(s, 8 * _CE)], sem)]


def _body(pts_hbm, tab_hbm, gs_hbm, out_hbm,
          pts_v, gs_v, w_v, idx_v, emb_v, out_v, sem0, sem1):
    cid = lax.axis_index("c")
    sid = lax.axis_index("s")
    wid = sid * _NC + cid
    pbase = wid * _PPW
    for d in range(3):
        pltpu.sync_copy(pts_hbm.at[pl.ds(d * _N + pbase, _PPW)],
                        pts_v.at[pl.ds(d * _PPW, _PPW)])
    pltpu.sync_copy(gs_hbm, gs_v)

    def _wait_buf(buf, sem):
        s = buf * 8 * _CE
        pltpu.make_async_copy(tab_hbm.at[idx_v.at[pl.ds(s, 8 * _CE)]],
                              emb_v.at[pl.ds(s, 8 * _CE)], sem).wait()

    @pl.loop(0, _NCHUNK * (_N_LEVELS // 2))
    def _step(s):
        lp = s & 7
        coff = (s >> 3) * _CHUNK
        l0 = lp * 2
        l1 = l0 + 1
        lprev = jnp.where(lp == 0, _N_LEVELS - 1, l1 - 2)
        _hash_phase(pts_v, gs_v, w_v, idx_v, 0, l0, coff)
        _fire_gathers(tab_hbm, idx_v, emb_v, 0, sem0)

        @pl.when(s > 0)
        def _():
            _wait_buf(1, sem1)
            _interp_phase(w_v, emb_v, out_v, 1, lprev)

            @pl.when(lp == 0)
            def _():
                pltpu.sync_copy(
                    out_v,
                    out_hbm.at[pl.ds((pbase + coff - _CHUNK) * 32, _CHUNK * 32)])

        _hash_phase(pts_v, gs_v, w_v, idx_v, 1, l1, coff)
        _fire_gathers(tab_hbm, idx_v, emb_v, 1, sem1)
        _wait_buf(0, sem0)
        _interp_phase(w_v, emb_v, out_v, 0, l0)

    _wait_buf(1, sem1)
    _interp_phase(w_v, emb_v, out_v, 1, _N_LEVELS - 1)
    pltpu.sync_copy(
        out_v,
        out_hbm.at[pl.ds((pbase + (_NCHUNK - 1) * _CHUNK) * 32, _CHUNK * 32)])


@jax.jit
def _embed(pts_flat, tab_flat, gs_splat):
    mesh = plsc.VectorSubcoreMesh(core_axis_name="c", subcore_axis_name="s")
    f = pl.kernel(
        _body,
        out_type=jax.ShapeDtypeStruct((_N * 32,), jnp.float32),
        mesh=mesh,
        scratch_types=[
            pltpu.VMEM((3 * _PPW,), jnp.float32),
            pltpu.VMEM((2 * _N_LEVELS * 16,), jnp.float32),
            pltpu.VMEM((2 * 3 * _CE,), jnp.float32),
            pltpu.VMEM((2 * 8 * _CE,), jnp.int32),
            pltpu.VMEM((2 * 8 * _CE,), jnp.float32),
            pltpu.VMEM((_CHUNK * 32,), jnp.float32),
            pltpu.SemaphoreType.DMA,
            pltpu.SemaphoreType.DMA,
        ],
        compiler_params=pltpu.CompilerParams(
            needs_layout_passes=False, use_tc_tiling_on_sc=False),
    )
    return f(pts_flat, tab_flat, gs_splat)


def kernel(input_points, tables):
    pts_flat = input_points.T.reshape(-1)
    tab_flat = (tables.reshape(_N_LEVELS, _TBL // 128, 128, _N_FEATS)
                .transpose(0, 1, 3, 2).reshape(-1))
    rows = []
    rrows = []
    for i in range(_N_LEVELS):
        res = jnp.floor(jnp.float32(_COARSE_RES) * jnp.float32(_B_GROWTH) ** i)
        gs = (jnp.float32(1.0) - jnp.float32(0.0)) / res
        rows.append(jnp.full((16,), gs, jnp.float32))
        rrows.append(jnp.full((16,), res, jnp.float32))
    gs_splat = jnp.concatenate(rows + rrows)
    out = _embed(pts_flat, tab_flat, gs_splat)
    return out.reshape(_N, 32)


# speedup vs baseline: 106.6554x; 1.0015x over previous
import numpy as np
import jax
import jax.numpy as jnp
from jax import lax
from jax.experimental import pallas as pl
from jax.experimental.pallas import tpu as pltpu
from jax.experimental.pallas import tpu_sc as plsc

_N_LEVELS = 16
_N_FEATS = 2
_FINEST_RES = 512.0
_COARSE_RES = 16.0
_LOG2_HASH_SZ = 19
_TBL = 1 << _LOG2_HASH_SZ
_MASK = _TBL - 1
_N = 262144
_B_GROWTH = float(np.exp((np.log(_FINEST_RES) - np.log(_COARSE_RES)) / (_N_LEVELS - 1)))
_CX, _CY, _CZ = np.int32(73856093), np.int32(19349663), np.int32(83492791)

_NC, _NS, _L = 2, 16, 16
_NW = _NC * _NS
_PPW = _N // _NW
_CHUNK = 128
_NCHUNK = _PPW // _CHUNK
_G16 = _CHUNK // 16
_G8 = _CHUNK // 8
_CE = _CHUNK * 2


def _expand(v, pairh):
    lo = v.at[pairh].get(mode="promise_in_bounds")
    hi = v.at[pairh + 8].get(mode="promise_in_bounds")
    return lo, hi


def _hash_phase(pts_v, gs_v, w_v, idx_v, buf, lvl, coff):
    lanes = lax.iota(jnp.int32, 16)
    pairh = lanes >> 1
    fpair = lanes & 1
    gs = gs_v[pl.ds(lvl * 16, 16)]
    rs = gs_v[pl.ds((16 + lvl) * 16, 16)]
    addv = fpair * 128 + lvl * (_N_FEATS * _TBL)
    for g in range(_G16):
        off = coff + g * 16
        p = [pts_v[pl.ds(d * _PPW + off, 16)] for d in range(3)]
        q = [p[d] * rs for d in range(3)]
        bl = [q[d].astype(jnp.int32) for d in range(3)]
        blf = [bl[d].astype(jnp.float32) for d in range(3)]
        vmin = [blf[d] * gs for d in range(3)]
        for d in range(3):
            w = (p[d] - vmin[d]) * rs
            wlo, whi = _expand(w, pairh)
            wb = (buf * 3 + d) * _CE + g * 32
            w_v[pl.ds(wb, 16)] = wlo
            w_v[pl.ds(wb + 16, 16)] = whi
        hx = bl[0] * _CX
        hy = bl[1] * _CY
        hz = bl[2] * _CZ
        hx1 = hx + _CX
        hy1 = hy + _CY
        hz1 = hz + _CZ
        k = 0
        for hxx in (hx, hx1):
            for hyy in (hy, hy1):
                hxy = hxx ^ hyy
                for hzz in (hz, hz1):
                    h = (hxy ^ hzz) & _MASK
                    hlo, hhi = _expand(h, pairh)
                    ib = (buf * 8 + k) * _CE + g * 32
                    idx_v[pl.ds(ib, 16)] = hlo + (hlo & -128) + addv
                    idx_v[pl.ds(ib + 16, 16)] = hhi + (hhi & -128) + addv
                    k += 1


def _interp_phase(w_v, emb_v, out_v, buf, lvl):
    lanes = lax.iota(jnp.int32, 16)
    pairh = lanes >> 1
    fpair = lanes & 1
    outp = pairh * 32 + fpair
    obase = lvl * 2
    for g in range(_G8):
        eb = (buf * 8) * _CE + g * 16
        e = [emb_v[pl.ds(eb + k * _CE, 16)] for k in range(8)]
        wb = (buf * 3) * _CE + g * 16
        wx = w_v[pl.ds(wb, 16)]
        wy = w_v[pl.ds(wb + _CE, 16)]
        wz = w_v[pl.ds(wb + 2 * _CE, 16)]
        omx = 1.0 - wx
        omy = 1.0 - wy
        omz = 1.0 - wz
        c00 = e[0] * omx + e[4] * wx
        c01 = e[1] * omx + e[5] * wx
        c10 = e[2] * omx + e[6] * wx
        c11 = e[3] * omx + e[7] * wx
        c0 = c00 * omy + c10 * wy
        c1 = c01 * omy + c11 * wy
        c = c0 * omz + c1 * wz
        oidx = outp + (g * 8 * 32 + obase)
        plsc.store_scatter(out_v, [oidx], c)


def _fire_gathers(tab_hbm, idx_v, emb_v, buf, sem):
    for h in range(2):
        s = buf * 8 * _CE + h * 4 * _CE
        pltpu.async_copy(tab_hbm.at[idx_v.at[pl.ds(s, 4 * _CE)]],
                         emb_v.at[pl.ds(s, 4 * _CE)], sem)


def _body(pts_hbm, tab_hbm, gs_hbm, out_hbm,
          pts_v, gs_v, w_v, idx_v, emb_v, out_v, sem0, sem1):
    cid = lax.axis_index("c")
    sid = lax.axis_index("s")
    wid = sid * _NC + cid
    pbase = wid * _PPW
    for d in range(3):
        pltpu.sync_copy(pts_hbm.at[pl.ds(d * _N + pbase, _PPW)],
                        pts_v.at[pl.ds(d * _PPW, _PPW)])
    pltpu.sync_copy(gs_hbm, gs_v)

    def _wait_buf(buf, sem):
        for h in range(2):
            s = buf * 8 * _CE + h * 4 * _CE
            pltpu.make_async_copy(tab_hbm.at[idx_v.at[pl.ds(s, 4 * _CE)]],
                                  emb_v.at[pl.ds(s, 4 * _CE)], sem).wait()

    @pl.loop(0, _NCHUNK * (_N_LEVELS // 2))
    def _step(s):
        lp = s & 7
        coff = (s >> 3) * _CHUNK
        l0 = lp * 2
        l1 = l0 + 1
        lprev = jnp.where(lp == 0, _N_LEVELS - 1, l1 - 2)
        _hash_phase(pts_v, gs_v, w_v, idx_v, 0, l0, coff)
        _fire_gathers(tab_hbm, idx_v, emb_v, 0, sem0)

        @pl.when(s > 0)
        def _():
            _wait_buf(1, sem1)
            _interp_phase(w_v, emb_v, out_v, 1, lprev)

            @pl.when(lp == 0)
            def _():
                pltpu.sync_copy(
                    out_v,
                    out_hbm.at[pl.ds((pbase + coff - _CHUNK) * 32, _CHUNK * 32)])

        _hash_phase(pts_v, gs_v, w_v, idx_v, 1, l1, coff)
        _fire_gathers(tab_hbm, idx_v, emb_v, 1, sem1)
        _wait_buf(0, sem0)
        _interp_phase(w_v, emb_v, out_v, 0, l0)

    _wait_buf(1, sem1)
    _interp_phase(w_v, emb_v, out_v, 1, _N_LEVELS - 1)
    pltpu.sync_copy(
        out_v,
        out_hbm.at[pl.ds((pbase + (_NCHUNK - 1) * _CHUNK) * 32, _CHUNK * 32)])


@jax.jit
def _embed(pts_flat, tab_flat, gs_splat):
    mesh = plsc.VectorSubcoreMesh(core_axis_name="c", subcore_axis_name="s")
    f = pl.kernel(
        _body,
        out_type=jax.ShapeDtypeStruct((_N * 32,), jnp.float32),
        mesh=mesh,
        scratch_types=[
            pltpu.VMEM((3 * _PPW,), jnp.float32),
            pltpu.VMEM((2 * _N_LEVELS * 16,), jnp.float32),
            pltpu.VMEM((2 * 3 * _CE,), jnp.float32),
            pltpu.VMEM((2 * 8 * _CE,), jnp.int32),
            pltpu.VMEM((2 * 8 * _CE,), jnp.float32),
            pltpu.VMEM((_CHUNK * 32,), jnp.float32),
            pltpu.SemaphoreType.DMA,
            pltpu.SemaphoreType.DMA,
        ],
        compiler_params=pltpu.CompilerParams(
            needs_layout_passes=False, use_tc_tiling_on_sc=False),
    )
    return f(pts_flat, tab_flat, gs_splat)


def kernel(input_points, tables):
    pts_flat = input_points.T.reshape(-1)
    tab_flat = (tables.reshape(_N_LEVELS, _TBL // 128, 128, _N_FEATS)
                .transpose(0, 1, 3, 2).reshape(-1))
    rows = []
    rrows = []
    for i in range(_N_LEVELS):
        res = jnp.floor(jnp.float32(_COARSE_RES) * jnp.float32(_B_GROWTH) ** i)
        gs = (jnp.float32(1.0) - jnp.float32(0.0)) / res
        rows.append(jnp.full((16,), gs, jnp.float32))
        rrows.append(jnp.full((16,), res, jnp.float32))
    gs_splat = jnp.concatenate(rows + rrows)
    out = _embed(pts_flat, tab_flat, gs_splat)
    return out.reshape(_N, 32)
